# Initial kernel scaffold; baseline (speedup 1.0000x reference)
import jax
import jax.numpy as jnp
from jax.experimental import pallas as pl


def kernel(x, edge_index, W1, b1, W2, b2):
    raise NotImplementedError("write your pallas kernel here")


# speedup vs baseline: 2.6572x; 2.6572x over previous
import functools

import jax
import jax.numpy as jnp
from jax import lax
from jax.experimental import pallas as pl
from jax.experimental.pallas import tpu as pltpu
from jax.experimental.pallas import tpu_sc as plsc

N = 10000
D = 256
H = 128
NPAD = 10240
EPAD = 163840
CH = 128
EPT = EPAD // 16
NCHUNK = EPT // CH
RPT = NPAD // 16
RB = 1000


def _norm(x):
    return jnp.maximum(jnp.sqrt(jnp.sum(x * x, axis=-1, keepdims=True)), 1e-15)


def _artanh(x):
    z = jnp.clip(x, -1.0 + 1e-7, 1.0 - 1e-7)
    return 0.5 * jnp.log((1.0 + z) / (1.0 - z))


def _proj(x):
    n = _norm(x)
    maxnorm = 1.0 - 1e-5
    return jnp.where(n > maxnorm, x / n * maxnorm, x)


def _expmap0(u):
    n = _norm(u)
    return jnp.tanh(n) * u / n


def _logmap0(x):
    n = _norm(x)
    return _artanh(n) * x / n


def _mobius_add(x, y):
    x2 = jnp.sum(x * x, axis=-1, keepdims=True)
    y2 = jnp.sum(y * y, axis=-1, keepdims=True)
    xy = jnp.sum(x * y, axis=-1, keepdims=True)
    num = (1.0 + 2.0 * xy + y2) * x + (1.0 - x2) * y
    den = 1.0 + 2.0 * xy + x2 * y2
    return num / jnp.maximum(den, 1e-15)


def _matvec_bias_tangent(h, wt, b):
    u = _logmap0(h)
    v = jnp.dot(u, wt, preferred_element_type=jnp.float32,
                precision=lax.Precision.HIGHEST)
    h1 = _proj(_expmap0(v))
    hb = _proj(_expmap0(b))
    h2 = _proj(_mobius_add(h1, hb))
    return _logmap0(h2)


def _tc_pre_body(x_ref, wt_ref, b_ref, o_ref):
    h = _proj(_expmap0(x_ref[...]))
    xt = _matvec_bias_tangent(h, wt_ref[...], b_ref[...])
    o_ref[0] = xt[:, :H]
    o_ref[1] = xt[:, H:]


def _agg_epilogue(agg_ref, xt_ref, deg_ref):
    s0 = agg_ref[0] + xt_ref[0]
    s1 = agg_ref[1] + xt_ref[1]
    s = jnp.concatenate([s0, s1], axis=-1)
    agg = s / (deg_ref[...] + 1.0)
    h = _proj(_expmap0(agg))
    t = jnp.maximum(_logmap0(h), 0.0)
    return _proj(_expmap0(t))


def _tc_mid_body(agg_ref, xt_ref, deg_ref, wt_ref, b_ref, o_ref):
    h = _agg_epilogue(agg_ref, xt_ref, deg_ref)
    xt = _matvec_bias_tangent(h, wt_ref[...], b_ref[...])
    o_ref[0] = xt[:, :H]
    o_ref[1] = xt[:, H:]


def _tc_post_body(agg_ref, xt_ref, deg_ref, o_ref):
    o_ref[...] = _agg_epilogue(agg_ref, xt_ref, deg_ref)


_SPLIT_SPEC = pl.BlockSpec((2, RB, H), lambda i: (0, i, 0))
_FULL_W = pl.BlockSpec((D, D), lambda i: (0, 0))
_FULL_B = pl.BlockSpec((1, D), lambda i: (0, 0))
_DEG_SPEC = pl.BlockSpec((RB, 1), lambda i: (i, 0))


def _tc_pre(x, wt, b):
    return pl.pallas_call(
        _tc_pre_body,
        grid=(N // RB,),
        in_specs=[pl.BlockSpec((RB, D), lambda i: (i, 0)), _FULL_W, _FULL_B],
        out_specs=_SPLIT_SPEC,
        out_shape=jax.ShapeDtypeStruct((2, N, H), jnp.float32),
    )(x, wt, b)


def _tc_mid(aggsum, xt, deg, wt, b):
    return pl.pallas_call(
        _tc_mid_body,
        grid=(N // RB,),
        in_specs=[_SPLIT_SPEC, _SPLIT_SPEC, _DEG_SPEC, _FULL_W, _FULL_B],
        out_specs=_SPLIT_SPEC,
        out_shape=jax.ShapeDtypeStruct((2, N, H), jnp.float32),
    )(aggsum, xt, deg, wt, b)


def _tc_post(aggsum, xt, deg):
    return pl.pallas_call(
        _tc_post_body,
        grid=(N // RB,),
        in_specs=[_SPLIT_SPEC, _SPLIT_SPEC, _DEG_SPEC],
        out_specs=pl.BlockSpec((RB, D), lambda i: (i, 0)),
        out_shape=jax.ShapeDtypeStruct((N, D), jnp.float32),
    )(aggsum, xt, deg)


def _sc_body_common(xt_hbm, src_hbm, dst_hbm, out_hbm, deg_hbm,
                    sidx, didx, rows, ones, zd, acc, dega, sem,
                    compute_deg):
    c = lax.axis_index("c")
    s = lax.axis_index("s")

    def _zrow(i, _):
        def _zcol(j, _):
            rows[i, pl.ds(j * 16, 16)] = jnp.zeros((16,), jnp.float32)
            return 0
        return lax.fori_loop(0, H // 16, _zcol, 0)
    lax.fori_loop(0, CH, _zrow, 0)
    for j in range(RPT // CH):
        pltpu.sync_copy(rows, acc.at[pl.ds(s * RPT + j * CH, CH)])

    if compute_deg:
        def _zd(i, _):
            zd[pl.ds(i * 16, 16)] = jnp.zeros((16,), jnp.float32)
            return 0
        lax.fori_loop(0, RPT // 16, _zd, 0)
        def _one(i, _):
            ones[pl.ds(i * 16, 16)] = jnp.ones((16,), jnp.float32)
            return 0
        lax.fori_loop(0, CH // 16, _one, 0)

        @pl.when(c == 0)
        def _():
            pltpu.sync_copy(zd, dega.at[pl.ds(s * RPT, RPT)])

    plsc.subcore_barrier()

    ebase = s * EPT
    coff = c * N

    def _chunk(j, _):
        off = ebase + j * CH
        pltpu.sync_copy(src_hbm.at[pl.ds(off, CH)], sidx)
        pltpu.sync_copy(dst_hbm.at[pl.ds(off, CH)], didx)
        for k in range(CH // 16):
            sidx[pl.ds(k * 16, 16)] = sidx[pl.ds(k * 16, 16)] + coff
        pltpu.async_copy(xt_hbm.at[sidx], rows, sem).wait()
        pltpu.sync_copy(rows, acc.at[didx], add=True)
        if compute_deg:
            @pl.when(c == 0)
            def _():
                pltpu.sync_copy(ones, dega.at[didx], add=True)
        return 0

    lax.fori_loop(0, NCHUNK, _chunk, 0)

    plsc.subcore_barrier()

    out_base = c * NPAD + s * RPT
    for j in range(RPT // CH):
        pltpu.sync_copy(acc.at[pl.ds(s * RPT + j * CH, CH)],
                        out_hbm.at[pl.ds(out_base + j * CH, CH)])
    if compute_deg:
        @pl.when(c == 0)
        def _():
            pltpu.sync_copy(dega.at[pl.ds(s * RPT, RPT)],
                            deg_hbm.at[pl.ds(s * RPT, RPT)])


def _sc_body_deg(xt_hbm, src_hbm, dst_hbm, out_hbm, deg_hbm,
                 sidx, didx, rows, ones, zd, acc, dega, sem):
    _sc_body_common(xt_hbm, src_hbm, dst_hbm, out_hbm, deg_hbm,
                    sidx, didx, rows, ones, zd, acc, dega, sem, True)


def _sc_body_nodeg(xt_hbm, src_hbm, dst_hbm, out_hbm,
                   sidx, didx, rows, ones, zd, acc, dega, sem):
    _sc_body_common(xt_hbm, src_hbm, dst_hbm, out_hbm, None,
                    sidx, didx, rows, ones, zd, acc, dega, sem, False)


@functools.lru_cache(maxsize=None)
def _make_sc(compute_deg):
    mesh = plsc.VectorSubcoreMesh(core_axis_name="c", subcore_axis_name="s",
                                  num_cores=2, num_subcores=16)
    if compute_deg:
        out_type = (jax.ShapeDtypeStruct((2 * NPAD, H), jnp.float32),
                    jax.ShapeDtypeStruct((NPAD,), jnp.float32))
        body = _sc_body_deg
    else:
        out_type = jax.ShapeDtypeStruct((2 * NPAD, H), jnp.float32)
        body = _sc_body_nodeg
    return pl.kernel(
        body,
        out_type=out_type,
        mesh=mesh,
        scratch_types=[
            pltpu.VMEM((CH,), jnp.int32),
            pltpu.VMEM((CH,), jnp.int32),
            pltpu.VMEM((CH, H), jnp.float32),
            pltpu.VMEM((CH,), jnp.float32),
            pltpu.VMEM((RPT,), jnp.float32),
            pltpu.VMEM_SHARED((NPAD, H), jnp.float32),
            pltpu.VMEM_SHARED((NPAD,), jnp.float32),
            pltpu.SemaphoreType.DMA,
        ],
    )


def kernel(x, edge_index, W1, b1, W2, b2):
    E = edge_index.shape[1]
    src = jnp.pad(edge_index[0], (0, EPAD - E))
    dst = jnp.pad(edge_index[1], (0, EPAD - E), constant_values=N)

    xt1 = _tc_pre(x, W1.T, b1[None, :])
    agg1, deg = _make_sc(True)(xt1.reshape(2 * N, H), src, dst)
    deg = deg.reshape(NPAD, 1)
    xt2 = _tc_mid(agg1.reshape(2, NPAD, H), xt1, deg, W2.T, b2[None, :])
    agg2 = _make_sc(False)(xt2.reshape(2 * N, H), src, dst)
    return _tc_post(agg2.reshape(2, NPAD, H), xt2, deg)


# speedup vs baseline: 3.5607x; 1.3400x over previous
import functools

import jax
import jax.numpy as jnp
from jax import lax
from jax.experimental import pallas as pl
from jax.experimental.pallas import tpu as pltpu
from jax.experimental.pallas import tpu_sc as plsc

N = 10000
D = 256
H = 128
NPAD = 10240
EPAD = 163840
CH = 128
EPT = EPAD // 16
NCHUNK = EPT // CH
RPT = NPAD // 16
RB = 1000


def _norm(x):
    return jnp.maximum(jnp.sqrt(jnp.sum(x * x, axis=-1, keepdims=True)), 1e-15)


def _artanh(x):
    z = jnp.clip(x, -1.0 + 1e-7, 1.0 - 1e-7)
    return 0.5 * jnp.log((1.0 + z) / (1.0 - z))


def _proj(x):
    n = _norm(x)
    maxnorm = 1.0 - 1e-5
    return jnp.where(n > maxnorm, x / n * maxnorm, x)


def _expmap0(u):
    n = _norm(u)
    return jnp.tanh(n) * u / n


def _logmap0(x):
    n = _norm(x)
    return _artanh(n) * x / n


def _mobius_add(x, y):
    x2 = jnp.sum(x * x, axis=-1, keepdims=True)
    y2 = jnp.sum(y * y, axis=-1, keepdims=True)
    xy = jnp.sum(x * y, axis=-1, keepdims=True)
    num = (1.0 + 2.0 * xy + y2) * x + (1.0 - x2) * y
    den = 1.0 + 2.0 * xy + x2 * y2
    return num / jnp.maximum(den, 1e-15)


def _matvec_bias_tangent(h, wt, b):
    u = _logmap0(h)
    v = jnp.dot(u, wt, preferred_element_type=jnp.float32,
                precision=lax.Precision.HIGHEST)
    h1 = _proj(_expmap0(v))
    hb = _proj(_expmap0(b))
    h2 = _proj(_mobius_add(h1, hb))
    return _logmap0(h2)


def _tc_pre_body(x_ref, wt_ref, b_ref, o_ref):
    h = _proj(_expmap0(x_ref[...]))
    xt = _matvec_bias_tangent(h, wt_ref[...], b_ref[...])
    o_ref[0] = xt[:, :H]
    o_ref[1] = xt[:, H:]


def _agg_epilogue(agg_ref, xt_ref, deg_ref):
    s0 = agg_ref[0] + xt_ref[0]
    s1 = agg_ref[1] + xt_ref[1]
    s = jnp.concatenate([s0, s1], axis=-1)
    agg = s / (deg_ref[...] + 1.0)
    h = _proj(_expmap0(agg))
    t = jnp.maximum(_logmap0(h), 0.0)
    return _proj(_expmap0(t))


def _tc_mid_body(agg_ref, xt_ref, deg_ref, wt_ref, b_ref, o_ref):
    h = _agg_epilogue(agg_ref, xt_ref, deg_ref)
    xt = _matvec_bias_tangent(h, wt_ref[...], b_ref[...])
    o_ref[0] = xt[:, :H]
    o_ref[1] = xt[:, H:]


def _tc_post_body(agg_ref, xt_ref, deg_ref, o_ref):
    o_ref[...] = _agg_epilogue(agg_ref, xt_ref, deg_ref)


_SPLIT_SPEC = pl.BlockSpec((2, RB, H), lambda i: (0, i, 0))
_FULL_W = pl.BlockSpec((D, D), lambda i: (0, 0))
_FULL_B = pl.BlockSpec((1, D), lambda i: (0, 0))
_DEG_SPEC = pl.BlockSpec((RB, 1), lambda i: (i, 0))


def _tc_pre(x, wt, b):
    return pl.pallas_call(
        _tc_pre_body,
        grid=(N // RB,),
        in_specs=[pl.BlockSpec((RB, D), lambda i: (i, 0)), _FULL_W, _FULL_B],
        out_specs=_SPLIT_SPEC,
        out_shape=jax.ShapeDtypeStruct((2, N, H), jnp.float32),
    )(x, wt, b)


def _tc_mid(aggsum, xt, deg, wt, b):
    return pl.pallas_call(
        _tc_mid_body,
        grid=(N // RB,),
        in_specs=[_SPLIT_SPEC, _SPLIT_SPEC, _DEG_SPEC, _FULL_W, _FULL_B],
        out_specs=_SPLIT_SPEC,
        out_shape=jax.ShapeDtypeStruct((2, N, H), jnp.float32),
    )(aggsum, xt, deg, wt, b)


def _tc_post(aggsum, xt, deg):
    return pl.pallas_call(
        _tc_post_body,
        grid=(N // RB,),
        in_specs=[_SPLIT_SPEC, _SPLIT_SPEC, _DEG_SPEC],
        out_specs=pl.BlockSpec((RB, D), lambda i: (i, 0)),
        out_shape=jax.ShapeDtypeStruct((N, D), jnp.float32),
    )(aggsum, xt, deg)


def _sc_body_common(xt_hbm, src_hbm, dst_hbm, out_hbm, deg_hbm,
                    sidxa, didxa, sidxb, didxb, rowsa, rowsb, ones, zd,
                    acc, dega, sema, semb, semga, semgb, compute_deg):
    c = lax.axis_index("c")
    s = lax.axis_index("s")

    def _zrow(i, _):
        def _zcol(j, _):
            rowsa[i, pl.ds(j * 16, 16)] = jnp.zeros((16,), jnp.float32)
            return 0
        return lax.fori_loop(0, H // 16, _zcol, 0)
    lax.fori_loop(0, CH, _zrow, 0)
    for j in range(RPT // CH):
        pltpu.sync_copy(rowsa, acc.at[pl.ds(s * RPT + j * CH, CH)])

    if compute_deg:
        def _zd(i, _):
            zd[pl.ds(i * 16, 16)] = jnp.zeros((16,), jnp.float32)
            return 0
        lax.fori_loop(0, RPT // 16, _zd, 0)
        def _one(i, _):
            ones[pl.ds(i * 16, 16)] = jnp.ones((16,), jnp.float32)
            return 0
        lax.fori_loop(0, CH // 16, _one, 0)

        @pl.when(c == 0)
        def _():
            pltpu.sync_copy(zd, dega.at[pl.ds(s * RPT, RPT)])

    plsc.subcore_barrier()

    ebase = s * EPT
    coff = c * N

    def _idx_issue(j, sidx, didx, sem):
        off = ebase + j * CH
        pltpu.async_copy(src_hbm.at[pl.ds(off, CH)], sidx, sem)
        pltpu.async_copy(dst_hbm.at[pl.ds(off, CH)], didx, sem)

    def _idx_wait(sidx, didx, sem):
        pltpu.make_async_copy(src_hbm.at[pl.ds(0, CH)], sidx, sem).wait()
        pltpu.make_async_copy(dst_hbm.at[pl.ds(0, CH)], didx, sem).wait()

    def _offset(sidx):
        for k in range(CH // 16):
            sidx[pl.ds(k * 16, 16)] = sidx[pl.ds(k * 16, 16)] + coff

    def _scatter(rows, didx):
        pltpu.sync_copy(rows, acc.at[didx], add=True)
        if compute_deg:
            @pl.when(c == 0)
            def _():
                pltpu.sync_copy(ones, dega.at[didx], add=True)

    _idx_issue(0, sidxa, didxa, sema)

    def _pair(i, _):
        _idx_wait(sidxa, didxa, sema)
        _offset(sidxa)
        ga = pltpu.async_copy(xt_hbm.at[sidxa], rowsa, semga)

        @pl.when(i > 0)
        def _():
            pltpu.make_async_copy(xt_hbm.at[sidxb], rowsb, semgb).wait()
            _scatter(rowsb, didxb)

        _idx_issue(2 * i + 1, sidxb, didxb, semb)
        _idx_wait(sidxb, didxb, semb)
        _offset(sidxb)
        pltpu.async_copy(xt_hbm.at[sidxb], rowsb, semgb)

        ga.wait()
        _scatter(rowsa, didxa)

        @pl.when(2 * i + 2 < NCHUNK)
        def _():
            _idx_issue(2 * i + 2, sidxa, didxa, sema)
        return 0

    lax.fori_loop(0, NCHUNK // 2, _pair, 0)
    pltpu.make_async_copy(xt_hbm.at[sidxb], rowsb, semgb).wait()
    _scatter(rowsb, didxb)

    plsc.subcore_barrier()

    out_base = c * NPAD + s * RPT
    for j in range(RPT // CH):
        pltpu.sync_copy(acc.at[pl.ds(s * RPT + j * CH, CH)],
                        out_hbm.at[pl.ds(out_base + j * CH, CH)])
    if compute_deg:
        @pl.when(c == 0)
        def _():
            pltpu.sync_copy(dega.at[pl.ds(s * RPT, RPT)],
                            deg_hbm.at[pl.ds(s * RPT, RPT)])


def _sc_body_deg(xt_hbm, src_hbm, dst_hbm, out_hbm, deg_hbm, *refs):
    _sc_body_common(xt_hbm, src_hbm, dst_hbm, out_hbm, deg_hbm, *refs,
                    compute_deg=True)


def _sc_body_nodeg(xt_hbm, src_hbm, dst_hbm, out_hbm, *refs):
    _sc_body_common(xt_hbm, src_hbm, dst_hbm, out_hbm, None, *refs,
                    compute_deg=False)


@functools.lru_cache(maxsize=None)
def _make_sc(compute_deg):
    mesh = plsc.VectorSubcoreMesh(core_axis_name="c", subcore_axis_name="s",
                                  num_cores=2, num_subcores=16)
    if compute_deg:
        out_type = (jax.ShapeDtypeStruct((2 * NPAD, H), jnp.float32),
                    jax.ShapeDtypeStruct((NPAD,), jnp.float32))
        body = _sc_body_deg
    else:
        out_type = jax.ShapeDtypeStruct((2 * NPAD, H), jnp.float32)
        body = _sc_body_nodeg
    return pl.kernel(
        body,
        out_type=out_type,
        mesh=mesh,
        scratch_types=[
            pltpu.VMEM((CH,), jnp.int32),
            pltpu.VMEM((CH,), jnp.int32),
            pltpu.VMEM((CH,), jnp.int32),
            pltpu.VMEM((CH,), jnp.int32),
            pltpu.VMEM((CH, H), jnp.float32),
            pltpu.VMEM((CH, H), jnp.float32),
            pltpu.VMEM((CH,), jnp.float32),
            pltpu.VMEM((RPT,), jnp.float32),
            pltpu.VMEM_SHARED((NPAD, H), jnp.float32),
            pltpu.VMEM_SHARED((NPAD,), jnp.float32),
            pltpu.SemaphoreType.DMA,
            pltpu.SemaphoreType.DMA,
            pltpu.SemaphoreType.DMA,
            pltpu.SemaphoreType.DMA,
        ],
    )


def kernel(x, edge_index, W1, b1, W2, b2):
    E = edge_index.shape[1]
    src = jnp.pad(edge_index[0], (0, EPAD - E))
    dst = jnp.pad(edge_index[1], (0, EPAD - E), constant_values=N)

    xt1 = _tc_pre(x, W1.T, b1[None, :])
    agg1, deg = _make_sc(True)(xt1.reshape(2 * N, H), src, dst)
    deg = deg.reshape(NPAD, 1)
    xt2 = _tc_mid(agg1.reshape(2, NPAD, H), xt1, deg, W2.T, b2[None, :])
    agg2 = _make_sc(False)(xt2.reshape(2 * N, H), src, dst)
    return _tc_post(agg2.reshape(2, NPAD, H), xt2, deg)


# speedup vs baseline: 3.6263x; 1.0184x over previous
import functools

import jax
import jax.numpy as jnp
from jax import lax
from jax.experimental import pallas as pl
from jax.experimental.pallas import tpu as pltpu
from jax.experimental.pallas import tpu_sc as plsc

N = 10000
D = 256
H = 128
NPAD = 10240
EPAD = 163840
CH = 128
EPT = EPAD // 16
NCHUNK = EPT // CH
RPT = NPAD // 16
RB = 1000


def _norm(x):
    return jnp.maximum(jnp.sqrt(jnp.sum(x * x, axis=-1, keepdims=True)), 1e-15)


def _artanh(x):
    z = jnp.clip(x, -1.0 + 1e-7, 1.0 - 1e-7)
    return 0.5 * jnp.log((1.0 + z) / (1.0 - z))


def _proj(x):
    n = _norm(x)
    maxnorm = 1.0 - 1e-5
    return jnp.where(n > maxnorm, x / n * maxnorm, x)


def _expmap0(u):
    n = _norm(u)
    return jnp.tanh(n) * u / n


def _logmap0(x):
    n = _norm(x)
    return _artanh(n) * x / n


def _mobius_add(x, y):
    x2 = jnp.sum(x * x, axis=-1, keepdims=True)
    y2 = jnp.sum(y * y, axis=-1, keepdims=True)
    xy = jnp.sum(x * y, axis=-1, keepdims=True)
    num = (1.0 + 2.0 * xy + y2) * x + (1.0 - x2) * y
    den = 1.0 + 2.0 * xy + x2 * y2
    return num / jnp.maximum(den, 1e-15)


def _matvec_bias_tangent(h, wt, b):
    u = _logmap0(h)
    v = jnp.dot(u, wt, preferred_element_type=jnp.float32,
                precision=lax.Precision.HIGHEST)
    h1 = _proj(_expmap0(v))
    hb = _proj(_expmap0(b))
    h2 = _proj(_mobius_add(h1, hb))
    return _logmap0(h2)


def _tc_pre_body(x_ref, wt_ref, b_ref, o_ref):
    h = _proj(_expmap0(x_ref[...]))
    xt = _matvec_bias_tangent(h, wt_ref[...], b_ref[...])
    o_ref[0] = xt[:, :H]
    o_ref[1] = xt[:, H:]


def _agg_epilogue(agg_ref, xt_ref, deg_ref):
    s0 = agg_ref[0] + xt_ref[0]
    s1 = agg_ref[1] + xt_ref[1]
    s = jnp.concatenate([s0, s1], axis=-1)
    agg = s / (deg_ref[...] + 1.0)
    h = _proj(_expmap0(agg))
    t = jnp.maximum(_logmap0(h), 0.0)
    return _proj(_expmap0(t))


def _tc_mid_body(agg_ref, xt_ref, deg_ref, wt_ref, b_ref, o_ref):
    h = _agg_epilogue(agg_ref, xt_ref, deg_ref)
    xt = _matvec_bias_tangent(h, wt_ref[...], b_ref[...])
    o_ref[0] = xt[:, :H]
    o_ref[1] = xt[:, H:]


def _tc_post_body(agg_ref, xt_ref, deg_ref, o_ref):
    o_ref[...] = _agg_epilogue(agg_ref, xt_ref, deg_ref)


_SPLIT_SPEC = pl.BlockSpec((2, RB, H), lambda i: (0, i, 0))
_FULL_W = pl.BlockSpec((D, D), lambda i: (0, 0))
_FULL_B = pl.BlockSpec((1, D), lambda i: (0, 0))
_DEG_SPEC = pl.BlockSpec((RB, 1), lambda i: (i, 0))


def _tc_pre(x, wt, b):
    return pl.pallas_call(
        _tc_pre_body,
        grid=(N // RB,),
        in_specs=[pl.BlockSpec((RB, D), lambda i: (i, 0)), _FULL_W, _FULL_B],
        out_specs=_SPLIT_SPEC,
        out_shape=jax.ShapeDtypeStruct((2, N, H), jnp.float32),
    )(x, wt, b)


def _tc_mid(aggsum, xt, deg, wt, b):
    return pl.pallas_call(
        _tc_mid_body,
        grid=(N // RB,),
        in_specs=[_SPLIT_SPEC, _SPLIT_SPEC, _DEG_SPEC, _FULL_W, _FULL_B],
        out_specs=_SPLIT_SPEC,
        out_shape=jax.ShapeDtypeStruct((2, N, H), jnp.float32),
    )(aggsum, xt, deg, wt, b)


def _tc_post(aggsum, xt, deg):
    return pl.pallas_call(
        _tc_post_body,
        grid=(N // RB,),
        in_specs=[_SPLIT_SPEC, _SPLIT_SPEC, _DEG_SPEC],
        out_specs=pl.BlockSpec((RB, D), lambda i: (i, 0)),
        out_shape=jax.ShapeDtypeStruct((N, D), jnp.float32),
    )(aggsum, xt, deg)


def _sc_body_common(xt_hbm, src_hbm, dst_hbm, out_hbm, deg_hbm,
                    sidx, didxa, didxb, rowsa, rowsb, ones, zd,
                    acc, dega, semi, sema, semb, semga, semgb, compute_deg):
    c = lax.axis_index("c")
    s = lax.axis_index("s")

    ebase = s * EPT

    di = pltpu.async_copy(src_hbm.at[pl.ds(ebase, EPT)], sidx, semi)

    def _didx_issue(j, didx, sem):
        pltpu.async_copy(dst_hbm.at[pl.ds(ebase + j * CH, CH)], didx, sem)

    def _didx_wait(didx, sem):
        pltpu.make_async_copy(dst_hbm.at[pl.ds(0, CH)], didx, sem).wait()

    _didx_issue(0, didxa, sema)
    _didx_issue(1, didxb, semb)

    def _zrow(i, _):
        def _zcol(j, _):
            rowsa[i, pl.ds(j * 16, 16)] = jnp.zeros((16,), jnp.float32)
            return 0
        return lax.fori_loop(0, H // 16, _zcol, 0)
    lax.fori_loop(0, CH, _zrow, 0)
    for j in range(RPT // CH):
        pltpu.sync_copy(rowsa, acc.at[pl.ds(s * RPT + j * CH, CH)])

    if compute_deg:
        def _zd(i, _):
            zd[pl.ds(i * 16, 16)] = jnp.zeros((16,), jnp.float32)
            return 0
        lax.fori_loop(0, RPT // 16, _zd, 0)
        def _one(i, _):
            ones[pl.ds(i * 16, 16)] = jnp.ones((16,), jnp.float32)
            return 0
        lax.fori_loop(0, CH // 16, _one, 0)

        @pl.when(c == 0)
        def _():
            pltpu.sync_copy(zd, dega.at[pl.ds(s * RPT, RPT)])

    di.wait()
    coff = c * N

    def _off(k, _):
        sidx[pl.ds(k * 16, 16)] = sidx[pl.ds(k * 16, 16)] + coff
        return 0
    lax.fori_loop(0, EPT // 16, _off, 0)

    plsc.subcore_barrier()

    def _scatter(rows, didx):
        pltpu.sync_copy(rows, acc.at[didx], add=True)
        if compute_deg:
            @pl.when(c == 0)
            def _():
                pltpu.sync_copy(ones, dega.at[didx], add=True)

    def _gather(j, rows, sem):
        return pltpu.async_copy(
            xt_hbm.at[sidx.at[pl.ds(j * CH, CH)]], rows, sem)

    def _pair(i, _):
        a = 2 * i
        ga = _gather(a, rowsa, semga)

        @pl.when(i > 0)
        def _():
            pltpu.make_async_copy(
                xt_hbm.at[sidx.at[pl.ds(0, CH)]], rowsb, semgb).wait()
            _didx_wait(didxb, semb)
            _scatter(rowsb, didxb)
            _didx_issue(a + 1, didxb, semb)

        _gather(a + 1, rowsb, semgb)
        ga.wait()
        _didx_wait(didxa, sema)
        _scatter(rowsa, didxa)

        @pl.when(a + 2 < NCHUNK)
        def _():
            _didx_issue(a + 2, didxa, sema)
        return 0

    lax.fori_loop(0, NCHUNK // 2, _pair, 0)
    pltpu.make_async_copy(
        xt_hbm.at[sidx.at[pl.ds(0, CH)]], rowsb, semgb).wait()
    _didx_wait(didxb, semb)
    _scatter(rowsb, didxb)

    plsc.subcore_barrier()

    out_base = c * NPAD + s * RPT
    for j in range(RPT // CH):
        pltpu.sync_copy(acc.at[pl.ds(s * RPT + j * CH, CH)],
                        out_hbm.at[pl.ds(out_base + j * CH, CH)])
    if compute_deg:
        @pl.when(c == 0)
        def _():
            pltpu.sync_copy(dega.at[pl.ds(s * RPT, RPT)],
                            deg_hbm.at[pl.ds(s * RPT, RPT)])


def _sc_body_deg(xt_hbm, src_hbm, dst_hbm, out_hbm, deg_hbm, *refs):
    _sc_body_common(xt_hbm, src_hbm, dst_hbm, out_hbm, deg_hbm, *refs,
                    compute_deg=True)


def _sc_body_nodeg(xt_hbm, src_hbm, dst_hbm, out_hbm, *refs):
    _sc_body_common(xt_hbm, src_hbm, dst_hbm, out_hbm, None, *refs,
                    compute_deg=False)


@functools.lru_cache(maxsize=None)
def _make_sc(compute_deg):
    mesh = plsc.VectorSubcoreMesh(core_axis_name="c", subcore_axis_name="s",
                                  num_cores=2, num_subcores=16)
    if compute_deg:
        out_type = (jax.ShapeDtypeStruct((2 * NPAD, H), jnp.float32),
                    jax.ShapeDtypeStruct((NPAD,), jnp.float32))
        body = _sc_body_deg
    else:
        out_type = jax.ShapeDtypeStruct((2 * NPAD, H), jnp.float32)
        body = _sc_body_nodeg
    return pl.kernel(
        body,
        out_type=out_type,
        mesh=mesh,
        scratch_types=[
            pltpu.VMEM((EPT,), jnp.int32),
            pltpu.VMEM((CH,), jnp.int32),
            pltpu.VMEM((CH,), jnp.int32),
            pltpu.VMEM((CH, H), jnp.float32),
            pltpu.VMEM((CH, H), jnp.float32),
            pltpu.VMEM((CH,), jnp.float32),
            pltpu.VMEM((RPT,), jnp.float32),
            pltpu.VMEM_SHARED((NPAD, H), jnp.float32),
            pltpu.VMEM_SHARED((NPAD,), jnp.float32),
            pltpu.SemaphoreType.DMA,
            pltpu.SemaphoreType.DMA,
            pltpu.SemaphoreType.DMA,
            pltpu.SemaphoreType.DMA,
            pltpu.SemaphoreType.DMA,
        ],
    )


def kernel(x, edge_index, W1, b1, W2, b2):
    E = edge_index.shape[1]
    src = jnp.pad(edge_index[0], (0, EPAD - E))
    dst = jnp.pad(edge_index[1], (0, EPAD - E), constant_values=N)

    xt1 = _tc_pre(x, W1.T, b1[None, :])
    agg1, deg = _make_sc(True)(xt1.reshape(2 * N, H), src, dst)
    deg = deg.reshape(NPAD, 1)
    xt2 = _tc_mid(agg1.reshape(2, NPAD, H), xt1, deg, W2.T, b2[None, :])
    agg2 = _make_sc(False)(xt2.reshape(2 * N, H), src, dst)
    return _tc_post(agg2.reshape(2, NPAD, H), xt2, deg)


# speedup vs baseline: 3.6867x; 1.0167x over previous
import functools

import jax
import jax.numpy as jnp
from jax import lax
from jax.experimental import pallas as pl
from jax.experimental.pallas import tpu as pltpu
from jax.experimental.pallas import tpu_sc as plsc

N = 10000
D = 256
H = 128
NPAD = 10240
EPAD = 163840
CH = 80
NB = 4
EPT = EPAD // 16
NCHUNK = EPT // CH
RPT = NPAD // 16
ZCH = 128
RB = 1000


def _norm(x):
    return jnp.maximum(jnp.sqrt(jnp.sum(x * x, axis=-1, keepdims=True)), 1e-15)


def _artanh(x):
    z = jnp.clip(x, -1.0 + 1e-7, 1.0 - 1e-7)
    return 0.5 * jnp.log((1.0 + z) / (1.0 - z))


def _proj(x):
    n = _norm(x)
    maxnorm = 1.0 - 1e-5
    return jnp.where(n > maxnorm, x / n * maxnorm, x)


def _expmap0(u):
    n = _norm(u)
    return jnp.tanh(n) * u / n


def _logmap0(x):
    n = _norm(x)
    return _artanh(n) * x / n


def _mobius_add(x, y):
    x2 = jnp.sum(x * x, axis=-1, keepdims=True)
    y2 = jnp.sum(y * y, axis=-1, keepdims=True)
    xy = jnp.sum(x * y, axis=-1, keepdims=True)
    num = (1.0 + 2.0 * xy + y2) * x + (1.0 - x2) * y
    den = 1.0 + 2.0 * xy + x2 * y2
    return num / jnp.maximum(den, 1e-15)


def _matvec_bias_tangent(h, wt, b):
    u = _logmap0(h)
    v = jnp.dot(u, wt, preferred_element_type=jnp.float32,
                precision=lax.Precision.HIGHEST)
    h1 = _proj(_expmap0(v))
    hb = _proj(_expmap0(b))
    h2 = _proj(_mobius_add(h1, hb))
    return _logmap0(h2)


def _tc_pre_body(x_ref, wt_ref, b_ref, o_ref):
    h = _proj(_expmap0(x_ref[...]))
    xt = _matvec_bias_tangent(h, wt_ref[...], b_ref[...])
    o_ref[0] = xt[:, :H]
    o_ref[1] = xt[:, H:]


def _agg_epilogue(agg_ref, xt_ref, deg_ref):
    s0 = agg_ref[0] + xt_ref[0]
    s1 = agg_ref[1] + xt_ref[1]
    s = jnp.concatenate([s0, s1], axis=-1)
    agg = s / (deg_ref[...] + 1.0)
    h = _proj(_expmap0(agg))
    t = jnp.maximum(_logmap0(h), 0.0)
    return _proj(_expmap0(t))


def _tc_mid_body(agg_ref, xt_ref, deg_ref, wt_ref, b_ref, o_ref):
    h = _agg_epilogue(agg_ref, xt_ref, deg_ref)
    xt = _matvec_bias_tangent(h, wt_ref[...], b_ref[...])
    o_ref[0] = xt[:, :H]
    o_ref[1] = xt[:, H:]


def _tc_post_body(agg_ref, xt_ref, deg_ref, o_ref):
    o_ref[...] = _agg_epilogue(agg_ref, xt_ref, deg_ref)


_SPLIT_SPEC = pl.BlockSpec((2, RB, H), lambda i: (0, i, 0))
_FULL_W = pl.BlockSpec((D, D), lambda i: (0, 0))
_FULL_B = pl.BlockSpec((1, D), lambda i: (0, 0))
_DEG_SPEC = pl.BlockSpec((RB, 1), lambda i: (i, 0))


def _tc_pre(x, wt, b):
    return pl.pallas_call(
        _tc_pre_body,
        grid=(N // RB,),
        in_specs=[pl.BlockSpec((RB, D), lambda i: (i, 0)), _FULL_W, _FULL_B],
        out_specs=_SPLIT_SPEC,
        out_shape=jax.ShapeDtypeStruct((2, N, H), jnp.float32),
    )(x, wt, b)


def _tc_mid(aggsum, xt, deg, wt, b):
    return pl.pallas_call(
        _tc_mid_body,
        grid=(N // RB,),
        in_specs=[_SPLIT_SPEC, _SPLIT_SPEC, _DEG_SPEC, _FULL_W, _FULL_B],
        out_specs=_SPLIT_SPEC,
        out_shape=jax.ShapeDtypeStruct((2, N, H), jnp.float32),
    )(aggsum, xt, deg, wt, b)


def _tc_post(aggsum, xt, deg):
    return pl.pallas_call(
        _tc_post_body,
        grid=(N // RB,),
        in_specs=[_SPLIT_SPEC, _SPLIT_SPEC, _DEG_SPEC],
        out_specs=pl.BlockSpec((RB, D), lambda i: (i, 0)),
        out_shape=jax.ShapeDtypeStruct((N, D), jnp.float32),
    )(aggsum, xt, deg)


def _sc_body_common(xt_hbm, src_hbm, dst_hbm, out_hbm, deg_hbm,
                    sidx, didx, rows, ones, zd, acc, dega,
                    semi, semg, sems, compute_deg):
    c = lax.axis_index("c")
    s = lax.axis_index("s")
    ebase = s * EPT
    coff = c * N

    def _idx_issue(j, b):
        off = ebase + j * CH
        pltpu.async_copy(src_hbm.at[pl.ds(off, CH)], sidx[b], semi[b])
        pltpu.async_copy(dst_hbm.at[pl.ds(off, CH)], didx[b], semi[b])

    def _idx_wait(b):
        pltpu.make_async_copy(src_hbm.at[pl.ds(0, CH)], sidx[b],
                              semi[b]).wait()
        pltpu.make_async_copy(dst_hbm.at[pl.ds(0, CH)], didx[b],
                              semi[b]).wait()

    def _off(b):
        for k in range(CH // 16):
            sidx[b][pl.ds(k * 16, 16)] = sidx[b][pl.ds(k * 16, 16)] + coff

    def _gather_issue(b):
        pltpu.async_copy(xt_hbm.at[sidx[b]], rows[b], semg[b])

    def _gather_wait(b):
        pltpu.make_async_copy(xt_hbm.at[sidx[b]], rows[b], semg[b]).wait()

    def _scat_issue(b):
        pltpu.async_copy(rows[b], acc.at[didx[b]], sems[b], add=True)
        if compute_deg:
            @pl.when(c == 0)
            def _():
                pltpu.async_copy(ones, dega.at[didx[b]], sems[b], add=True)

    def _scat_wait(b):
        pltpu.make_async_copy(rows[b], acc.at[didx[b]], sems[b]).wait()
        if compute_deg:
            @pl.when(c == 0)
            def _():
                pltpu.make_async_copy(ones, dega.at[didx[b]], sems[b]).wait()

    for b in range(NB - 1):
        _idx_issue(b, b)

    def _zrow(i, _):
        def _zcol(j, _):
            rows[0][i, pl.ds(j * 16, 16)] = jnp.zeros((16,), jnp.float32)
            return 0
        return lax.fori_loop(0, H // 16, _zcol, 0)
    lax.fori_loop(0, CH, _zrow, 0)
    for j in range(RPT // CH):
        pltpu.sync_copy(rows[0], acc.at[pl.ds(s * RPT + j * CH, CH)])

    if compute_deg:
        def _zd(i, _):
            zd[pl.ds(i * 16, 16)] = jnp.zeros((16,), jnp.float32)
            return 0
        lax.fori_loop(0, RPT // 16, _zd, 0)
        def _one(i, _):
            ones[pl.ds(i * 16, 16)] = jnp.ones((16,), jnp.float32)
            return 0
        lax.fori_loop(0, CH // 16, _one, 0)

        @pl.when(c == 0)
        def _():
            pltpu.sync_copy(zd, dega.at[pl.ds(s * RPT, RPT)])

    for b in range(2):
        _idx_wait(b)
        _off(b)
        _gather_issue(b)

    plsc.subcore_barrier()

    def _group(q, _):
        for b in range(NB):
            j = NB * q + b
            b2 = (b + 2) % NB
            b3 = (b + 3) % NB
            _gather_wait(b)
            _scat_issue(b)

            @pl.when(j >= 1)
            def _():
                _scat_wait(b3)

            @pl.when(j + 3 < NCHUNK)
            def _():
                _idx_issue(j + 3, b3)

            @pl.when(j + 2 < NCHUNK)
            def _():
                _idx_wait(b2)
                _off(b2)
                _gather_issue(b2)
        return 0

    lax.fori_loop(0, NCHUNK // NB, _group, 0)
    _scat_wait((NCHUNK - 1) % NB)

    plsc.subcore_barrier()

    out_base = c * NPAD + s * RPT
    for j in range(RPT // CH):
        pltpu.sync_copy(acc.at[pl.ds(s * RPT + j * CH, CH)],
                        out_hbm.at[pl.ds(out_base + j * CH, CH)])
    if compute_deg:
        @pl.when(c == 0)
        def _():
            pltpu.sync_copy(dega.at[pl.ds(s * RPT, RPT)],
                            deg_hbm.at[pl.ds(s * RPT, RPT)])


def _unpack(r):
    return (list(r[0:4]), list(r[4:8]), list(r[8:12]), r[12], r[13],
            r[14], r[15], list(r[16:20]), list(r[20:24]), list(r[24:28]))


def _sc_body_deg(xt_hbm, src_hbm, dst_hbm, out_hbm, deg_hbm, *refs):
    _sc_body_common(xt_hbm, src_hbm, dst_hbm, out_hbm, deg_hbm,
                    *_unpack(refs), compute_deg=True)


def _sc_body_nodeg(xt_hbm, src_hbm, dst_hbm, out_hbm, *refs):
    _sc_body_common(xt_hbm, src_hbm, dst_hbm, out_hbm, None,
                    *_unpack(refs), compute_deg=False)


@functools.lru_cache(maxsize=None)
def _make_sc(compute_deg):
    mesh = plsc.VectorSubcoreMesh(core_axis_name="c", subcore_axis_name="s",
                                  num_cores=2, num_subcores=16)
    if compute_deg:
        out_type = (jax.ShapeDtypeStruct((2 * NPAD, H), jnp.float32),
                    jax.ShapeDtypeStruct((NPAD,), jnp.float32))
        body = _sc_body_deg
    else:
        out_type = jax.ShapeDtypeStruct((2 * NPAD, H), jnp.float32)
        body = _sc_body_nodeg
    return pl.kernel(
        body,
        out_type=out_type,
        mesh=mesh,
        scratch_types=(
            [pltpu.VMEM((CH,), jnp.int32) for _ in range(NB)]
            + [pltpu.VMEM((CH,), jnp.int32) for _ in range(NB)]
            + [pltpu.VMEM((CH, H), jnp.float32) for _ in range(NB)]
            + [
                pltpu.VMEM((CH,), jnp.float32),
                pltpu.VMEM((RPT,), jnp.float32),
                pltpu.VMEM_SHARED((NPAD, H), jnp.float32),
                pltpu.VMEM_SHARED((NPAD,), jnp.float32),
            ]
            + [pltpu.SemaphoreType.DMA for _ in range(3 * NB)]
        ),
    )


def kernel(x, edge_index, W1, b1, W2, b2):
    E = edge_index.shape[1]
    src = jnp.pad(edge_index[0], (0, EPAD - E))
    dst = jnp.pad(edge_index[1], (0, EPAD - E), constant_values=N)

    xt1 = _tc_pre(x, W1.T, b1[None, :])
    agg1, deg = _make_sc(True)(xt1.reshape(2 * N, H), src, dst)
    deg = deg.reshape(NPAD, 1)
    xt2 = _tc_mid(agg1.reshape(2, NPAD, H), xt1, deg, W2.T, b2[None, :])
    agg2 = _make_sc(False)(xt2.reshape(2 * N, H), src, dst)
    return _tc_post(agg2.reshape(2, NPAD, H), xt2, deg)


# speedup vs baseline: 3.6890x; 1.0006x over previous
import functools

import jax
import jax.numpy as jnp
from jax import lax
from jax.experimental import pallas as pl
from jax.experimental.pallas import tpu as pltpu
from jax.experimental.pallas import tpu_sc as plsc

N = 10000
D = 256
H = 128
NPAD = 10240
EPAD = 163840
CH = 80
NB = 4
EPT = EPAD // 16
NCHUNK = EPT // CH
RPT = NPAD // 16
RB = 1000


def _norm(x):
    return jnp.maximum(jnp.sqrt(jnp.sum(x * x, axis=-1, keepdims=True)), 1e-15)


def _artanh(x):
    z = jnp.clip(x, -1.0 + 1e-7, 1.0 - 1e-7)
    return 0.5 * jnp.log((1.0 + z) / (1.0 - z))


def _proj(x):
    n = _norm(x)
    maxnorm = 1.0 - 1e-5
    return jnp.where(n > maxnorm, x / n * maxnorm, x)


def _expmap0(u):
    n = _norm(u)
    return jnp.tanh(n) * u / n


def _logmap0(x):
    n = _norm(x)
    return _artanh(n) * x / n


def _mobius_add(x, y):
    x2 = jnp.sum(x * x, axis=-1, keepdims=True)
    y2 = jnp.sum(y * y, axis=-1, keepdims=True)
    xy = jnp.sum(x * y, axis=-1, keepdims=True)
    num = (1.0 + 2.0 * xy + y2) * x + (1.0 - x2) * y
    den = 1.0 + 2.0 * xy + x2 * y2
    return num / jnp.maximum(den, 1e-15)


def _matvec_bias_tangent(h, wt, b):
    u = _logmap0(h)
    v = jnp.dot(u, wt, preferred_element_type=jnp.float32,
                precision=lax.Precision.HIGHEST)
    h1 = _proj(_expmap0(v))
    hb = _proj(_expmap0(b))
    h2 = _proj(_mobius_add(h1, hb))
    return _logmap0(h2)


def _tc_pre_body(x_ref, wt_ref, b_ref, o_ref):
    h = _proj(_expmap0(x_ref[...]))
    xt = _matvec_bias_tangent(h, wt_ref[...], b_ref[...])
    o_ref[0] = xt[:, :H]
    o_ref[1] = xt[:, H:]


def _agg_epilogue(agg_ref, xt_ref, deg_ref):
    s0 = agg_ref[0] + xt_ref[0]
    s1 = agg_ref[1] + xt_ref[1]
    s = jnp.concatenate([s0, s1], axis=-1)
    agg = s / (deg_ref[...] + 1.0)
    h = _proj(_expmap0(agg))
    t = jnp.maximum(_logmap0(h), 0.0)
    return _proj(_expmap0(t))


def _tc_mid_body(agg_ref, xt_ref, deg_ref, wt_ref, b_ref, o_ref):
    h = _agg_epilogue(agg_ref, xt_ref, deg_ref)
    xt = _matvec_bias_tangent(h, wt_ref[...], b_ref[...])
    o_ref[0] = xt[:, :H]
    o_ref[1] = xt[:, H:]


def _tc_post_body(agg_ref, xt_ref, deg_ref, o_ref):
    o_ref[...] = _agg_epilogue(agg_ref, xt_ref, deg_ref)


_SPLIT_SPEC = pl.BlockSpec((2, RB, H), lambda i: (0, i, 0))
_FULL_W = pl.BlockSpec((D, D), lambda i: (0, 0))
_FULL_B = pl.BlockSpec((1, D), lambda i: (0, 0))
_DEG_SPEC = pl.BlockSpec((RB, 1), lambda i: (i, 0))


def _tc_pre(x, wt, b):
    return pl.pallas_call(
        _tc_pre_body,
        grid=(N // RB,),
        in_specs=[pl.BlockSpec((RB, D), lambda i: (i, 0)), _FULL_W, _FULL_B],
        out_specs=_SPLIT_SPEC,
        out_shape=jax.ShapeDtypeStruct((2, N, H), jnp.float32),
    )(x, wt, b)


def _tc_mid(aggsum, xt, deg, wt, b):
    return pl.pallas_call(
        _tc_mid_body,
        grid=(N // RB,),
        in_specs=[_SPLIT_SPEC, _SPLIT_SPEC, _DEG_SPEC, _FULL_W, _FULL_B],
        out_specs=_SPLIT_SPEC,
        out_shape=jax.ShapeDtypeStruct((2, N, H), jnp.float32),
    )(aggsum, xt, deg, wt, b)


def _tc_post(aggsum, xt, deg):
    return pl.pallas_call(
        _tc_post_body,
        grid=(N // RB,),
        in_specs=[_SPLIT_SPEC, _SPLIT_SPEC, _DEG_SPEC],
        out_specs=pl.BlockSpec((RB, D), lambda i: (i, 0)),
        out_shape=jax.ShapeDtypeStruct((N, D), jnp.float32),
    )(aggsum, xt, deg)


def _sc_body_common(xt_hbm, src_hbm, dst_hbm, out_hbm, deg_hbm,
                    sidx, didx, rows, ones, zd, acc, dega,
                    semi, semg, sems, compute_deg):
    c = lax.axis_index("c")
    s = lax.axis_index("s")
    ebase = s * EPT
    coff = c * N

    def _idx_issue(j, b):
        off = ebase + j * CH
        pltpu.async_copy(src_hbm.at[pl.ds(off, CH)], sidx[b], semi[b])
        pltpu.async_copy(dst_hbm.at[pl.ds(off, CH)], didx[b], semi[b])

    def _idx_wait(b):
        pltpu.make_async_copy(src_hbm.at[pl.ds(0, CH)], sidx[b],
                              semi[b]).wait()
        pltpu.make_async_copy(dst_hbm.at[pl.ds(0, CH)], didx[b],
                              semi[b]).wait()

    def _off(b):
        for k in range(CH // 16):
            sidx[b][pl.ds(k * 16, 16)] = sidx[b][pl.ds(k * 16, 16)] + coff

    def _gather_issue(b):
        pltpu.async_copy(xt_hbm.at[sidx[b]], rows[b], semg[b])

    def _gather_wait(b):
        pltpu.make_async_copy(xt_hbm.at[sidx[b]], rows[b], semg[b]).wait()

    def _scat_issue(b):
        pltpu.async_copy(rows[b], acc.at[didx[b]], sems[b], add=True)
        if compute_deg:
            @pl.when(c == 0)
            def _():
                pltpu.async_copy(ones, dega.at[didx[b]], sems[b], add=True)

    def _scat_wait(b):
        pltpu.make_async_copy(rows[b], acc.at[didx[b]], sems[b]).wait()
        if compute_deg:
            @pl.when(c == 0)
            def _():
                pltpu.make_async_copy(ones, dega.at[didx[b]], sems[b]).wait()

    for b in range(NB - 1):
        _idx_issue(b, b)

    def _zrow(i, _):
        def _zcol(j, _):
            rows[0][i, pl.ds(j * 16, 16)] = jnp.zeros((16,), jnp.float32)
            return 0
        return lax.fori_loop(0, H // 16, _zcol, 0)
    lax.fori_loop(0, CH, _zrow, 0)
    for j in range(RPT // CH):
        pltpu.sync_copy(rows[0], acc.at[pl.ds(s * RPT + j * CH, CH)])

    if compute_deg:
        def _zd(i, _):
            zd[pl.ds(i * 16, 16)] = jnp.zeros((16,), jnp.float32)
            return 0
        lax.fori_loop(0, RPT // 16, _zd, 0)
        def _one(i, _):
            ones[pl.ds(i * 16, 16)] = jnp.ones((16,), jnp.float32)
            return 0
        lax.fori_loop(0, CH // 16, _one, 0)

        @pl.when(c == 0)
        def _():
            pltpu.sync_copy(zd, dega.at[pl.ds(s * RPT, RPT)])

    for b in range(2):
        _idx_wait(b)
        _off(b)
        _gather_issue(b)

    plsc.subcore_barrier()

    def _group(q, _):
        for b in range(NB):
            j = NB * q + b
            b2 = (b + 2) % NB
            b3 = (b + 3) % NB
            _gather_wait(b)
            _scat_issue(b)

            @pl.when(j >= 1)
            def _():
                _scat_wait(b3)

            @pl.when(j + 3 < NCHUNK)
            def _():
                _idx_issue(j + 3, b3)

            @pl.when(j + 2 < NCHUNK)
            def _():
                _idx_wait(b2)
                _off(b2)
                _gather_issue(b2)
        return 0

    lax.fori_loop(0, NCHUNK // NB, _group, 0)
    _scat_wait((NCHUNK - 1) % NB)

    plsc.subcore_barrier()

    out_base = c * NPAD + s * RPT
    for j in range(RPT // CH):
        pltpu.sync_copy(acc.at[pl.ds(s * RPT + j * CH, CH)],
                        out_hbm.at[pl.ds(out_base + j * CH, CH)])
    if compute_deg:
        @pl.when(c == 0)
        def _():
            pltpu.sync_copy(dega.at[pl.ds(s * RPT, RPT)],
                            deg_hbm.at[pl.ds(s * RPT, RPT)])


def _unpack(r):
    return (list(r[0:4]), list(r[4:8]), list(r[8:12]), r[12], r[13],
            r[14], r[15], list(r[16:20]), list(r[20:24]), list(r[24:28]))


def _sc_body_deg(xt_hbm, src_hbm, dst_hbm, out_hbm, deg_hbm, *refs):
    _sc_body_common(xt_hbm, src_hbm, dst_hbm, out_hbm, deg_hbm,
                    *_unpack(refs), compute_deg=True)


def _sc_body_nodeg(xt_hbm, src_hbm, dst_hbm, out_hbm, *refs):
    _sc_body_common(xt_hbm, src_hbm, dst_hbm, out_hbm, None,
                    *_unpack(refs), compute_deg=False)


@functools.lru_cache(maxsize=None)
def _make_sc(compute_deg):
    mesh = plsc.VectorSubcoreMesh(core_axis_name="c", subcore_axis_name="s",
                                  num_cores=2, num_subcores=16)
    if compute_deg:
        out_type = (jax.ShapeDtypeStruct((2 * NPAD, H), jnp.float32),
                    jax.ShapeDtypeStruct((NPAD,), jnp.float32))
        body = _sc_body_deg
    else:
        out_type = jax.ShapeDtypeStruct((2 * NPAD, H), jnp.float32)
        body = _sc_body_nodeg
    return pl.kernel(
        body,
        out_type=out_type,
        mesh=mesh,
        scratch_types=(
            [pltpu.VMEM((CH,), jnp.int32) for _ in range(NB)]
            + [pltpu.VMEM((CH,), jnp.int32) for _ in range(NB)]
            + [pltpu.VMEM((CH, H), jnp.float32) for _ in range(NB)]
            + [
                pltpu.VMEM((CH,), jnp.float32),
                pltpu.VMEM((RPT,), jnp.float32),
                pltpu.VMEM_SHARED((NPAD, H), jnp.float32),
                pltpu.VMEM_SHARED((NPAD,), jnp.float32),
            ]
            + [pltpu.SemaphoreType.DMA for _ in range(3 * NB)]
        ),
    )


def kernel(x, edge_index, W1, b1, W2, b2):
    E = edge_index.shape[1]
    src = jnp.pad(edge_index[0], (0, EPAD - E))
    dst = jnp.pad(edge_index[1], (0, EPAD - E), constant_values=N)

    xt1 = _tc_pre(x, W1.T, b1[None, :])
    agg1, deg = _make_sc(True)(xt1.reshape(2 * N, H), src, dst)
    deg = deg.reshape(NPAD, 1)
    xt2 = _tc_mid(agg1.reshape(2, NPAD, H), xt1, deg, W2.T, b2[None, :])
    agg2 = _make_sc(False)(xt2.reshape(2 * N, H), src, dst)
    return _tc_post(agg2.reshape(2, NPAD, H), xt2, deg)
